# baseline (speedup 1.0000x reference)
import jax
import jax.numpy as jnp
from jax import lax
from jax.experimental import pallas as pl
from jax.experimental.pallas import tpu as pltpu
from jax.experimental.pallas import tpu_sc as plsc

_N = 10000
_E = 320000
_D = 128
_MASK_NUM = 1500
_REPL_NUM = 150
_NC = 2
_NS = 16
_NW = _NC * _NS
_EPW = 2 * _E // _NW
_RPW = 320
_LAST_BASE = _N - _RPW
_NPAD = _N + 16
_IMIN = jnp.iinfo(jnp.int32).min

def _op_constants():
    kg, kr = jax.random.split(jax.random.key(42))
    gum_pad = jnp.pad(
        jax.random.gumbel(kg, (_N,), dtype=jnp.float32),
        (0, _NPAD - _N)).reshape(1, _NPAD)
    rand_flat = jax.random.normal(
        kr, (_REPL_NUM, _D), dtype=jnp.float32).reshape(-1)
    return gum_pad, rand_flat


def _sget(ref, i):
    return ref[pl.ds(i, 16)][0]


def _bincount_body(edges_hbm, hist_hbm, ev, hv):
    c = lax.axis_index("c")
    s = lax.axis_index("s")
    wid = s * _NC + c
    base = wid * _EPW
    pltpu.sync_copy(edges_hbm.at[pl.ds(base, _EPW)], ev)
    z = jnp.zeros((16,), jnp.int32)

    @plsc.parallel_loop(0, _NPAD // 16, unroll=8)
    def _(i):
        hv[pl.ds(i * 16, 16)] = z

    ones = jnp.ones((16,), jnp.int32)

    @plsc.parallel_loop(0, _EPW // 16, unroll=8)
    def _(i):
        idx = ev[pl.ds(i * 16, 16)]
        plsc.addupdate_scatter(hv, [idx], ones)

    pltpu.sync_copy(hv, hist_hbm.at[wid, 0])


def _sc_bincount(edges):
    mesh = plsc.VectorSubcoreMesh(core_axis_name="c", subcore_axis_name="s")
    return pl.kernel(
        _bincount_body,
        out_type=jax.ShapeDtypeStruct((_NW, 1, _NPAD), jnp.int32),
        mesh=mesh,
        scratch_types=[
            pltpu.VMEM((_EPW,), jnp.int32),
            pltpu.VMEM((_NPAD,), jnp.int32),
        ],
        compiler_params=pltpu.CompilerParams(needs_layout_passes=False),
    )(edges)


def _tc_score_body(hist_ref, gum_ref, mask_ref, skey_ref, tz_ref, tr_ref):
    col = lax.broadcasted_iota(jnp.int32, (1, _NPAD), 1)
    valid = col < _N
    deg = jnp.sum(hist_ref[...], axis=0, keepdims=True).astype(jnp.float32)
    m = jnp.max(deg)
    e = jnp.where(valid, jnp.exp(deg - m), jnp.float32(0.0))
    ssum = jnp.sum(e)
    prob = e / ssum
    sc = jnp.log(prob + jnp.float32(1e-20)) + gum_ref[...]
    b = lax.bitcast_convert_type(sc, jnp.int32)
    u = jnp.where(
        b < 0,
        lax.bitcast_convert_type(~b, jnp.uint32),
        lax.bitcast_convert_type(b, jnp.uint32) | jnp.uint32(0x80000000),
    )
    u = jnp.where(valid, u, jnp.uint32(0))

    def bit_body(t, ps):
        p1, p2 = ps
        sh = jnp.uint32(31) - t.astype(jnp.uint32)
        one = jnp.uint32(1) << sh
        c1 = p1 | one
        c2 = p2 | one
        n1 = jnp.sum((u >= c1).astype(jnp.int32))
        n2 = jnp.sum((u >= c2).astype(jnp.int32))
        return (jnp.where(n1 >= _MASK_NUM, c1, p1),
                jnp.where(n2 >= _REPL_NUM, c2, p2))

    tz, tr = lax.fori_loop(0, 32, bit_body, (jnp.uint32(0), jnp.uint32(0)))
    flip = jnp.uint32(0x80000000)
    mask_ref[...] = (u >= tz)[:, :_N]
    skey_ref[...] = lax.bitcast_convert_type(u ^ flip, jnp.int32)
    tz_ref[...] = jnp.full(
        (1, 16), lax.bitcast_convert_type(tz ^ flip, jnp.int32), jnp.int32)
    tr_ref[...] = jnp.full(
        (1, 16), lax.bitcast_convert_type(tr ^ flip, jnp.int32), jnp.int32)


def _tc_scores(hist, gum2):
    return pl.pallas_call(
        _tc_score_body,
        out_shape=[
            jax.ShapeDtypeStruct((1, _N), jnp.bool_),
            jax.ShapeDtypeStruct((1, _NPAD), jnp.int32),
            jax.ShapeDtypeStruct((1, 16), jnp.int32),
            jax.ShapeDtypeStruct((1, 16), jnp.int32),
        ],
    )(hist, gum2)


def _sc_apply_body(x_hbm, skey_hbm, tz_hbm, tr_hbm, rand_hbm, out_hbm,
                   buf, ub, tzv, trv, ckey, cidx, lidx, olist, semx):
    c = lax.axis_index("c")
    s = lax.axis_index("s")
    wid = s * _NC + c
    base = jnp.minimum(wid * _RPW, _LAST_BASE)
    cpx = pltpu.async_copy(x_hbm.at[pl.ds(base * _D, _RPW * _D)], buf, semx)
    pltpu.sync_copy(skey_hbm, ub)
    pltpu.sync_copy(tz_hbm, tzv)
    pltpu.sync_copy(tr_hbm, trv)
    tz = tzv[...]
    tr = trv[...]
    lane = lax.iota(jnp.int32, 16)
    imin = jnp.full((16,), _IMIN, jnp.int32)
    for q in range(11):
        ckey[pl.ds(q * 16, 16)] = imin

    def zscan(i, cnt):
        v = ub[pl.ds(base + i * 16, 16)]
        msk = v >= tz
        pc = plsc.all_reduce_population_count(msk)[0]

        @pl.when(pc > 0)
        def _():
            lv = i * 16 + lane
            plsc.store_compressed(lidx.at[pl.ds(cnt, 16)], lv, mask=msk)

        return cnt + pc

    zcnt = lax.fori_loop(0, _RPW // 16, zscan, jnp.int32(0))

    def cscan(i, cnt):
        v = ub[pl.ds(i * 16, 16)]
        msk = v >= tr
        pc = plsc.all_reduce_population_count(msk)[0]

        @pl.when(pc > 0)
        def _():
            gv = i * 16 + lane
            plsc.store_compressed(ckey.at[pl.ds(cnt, 16)], v, mask=msk)
            plsc.store_compressed(cidx.at[pl.ds(cnt, 16)], gv, mask=msk)

        return cnt + pc

    ccnt = lax.fori_loop(0, _NPAD // 16, cscan, jnp.int32(0))

    def oscan(q, cnt):
        jv = q * 16 + lane
        ci = cidx[pl.ds(q * 16, 16)]
        own = (jv < ccnt) & (ci >= base) & (ci < base + _RPW)
        plsc.store_compressed(olist.at[pl.ds(cnt, 16)], jv, mask=own)
        return cnt + plsc.all_reduce_population_count(own)[0]

    ocnt = lax.fori_loop(0, 11, oscan, jnp.int32(0))

    cpx.wait()
    zrow = jnp.zeros((16,), jnp.float32)

    def zero_body(j, _):
        r = _sget(lidx, j)
        for cc in range(8):
            buf[pl.ds(r * _D + cc * 16, 16)] = zrow
        return 0

    lax.fori_loop(0, zcnt, zero_body, 0)

    def repl_body(jj, _):
        j = _sget(olist, jj)
        kj = _sget(ckey, j)
        gi = _sget(cidx, j)
        kjv = jnp.full((16,), kj, jnp.int32)
        giv = jnp.full((16,), gi, jnp.int32)

        def rk(q, r):
            ck = ckey[pl.ds(q * 16, 16)]
            ci = cidx[pl.ds(q * 16, 16)]
            cmp = (ck > kjv) | ((ck == kjv) & (ci < giv))
            return r + plsc.all_reduce_population_count(cmp)[0]

        rank = lax.fori_loop(0, 11, rk, jnp.int32(0))
        rank = jnp.minimum(rank, _REPL_NUM - 1)
        pltpu.sync_copy(rand_hbm.at[pl.ds(rank * _D, _D)],
                        buf.at[pl.ds((gi - base) * _D, _D)])
        return 0

    lax.fori_loop(0, ocnt, repl_body, 0)
    pltpu.sync_copy(buf, out_hbm.at[pl.ds(base * _D, _RPW * _D)])


def _sc_apply(x, skey_pad, tzv, trv, rand):
    mesh = plsc.VectorSubcoreMesh(core_axis_name="c", subcore_axis_name="s")
    return pl.kernel(
        _sc_apply_body,
        out_type=jax.ShapeDtypeStruct((_N * _D,), jnp.float32),
        mesh=mesh,
        scratch_types=[
            pltpu.VMEM((_RPW * _D,), jnp.float32),
            pltpu.VMEM((_NPAD,), jnp.int32),
            pltpu.VMEM((16,), jnp.int32),
            pltpu.VMEM((16,), jnp.int32),
            pltpu.VMEM((176,), jnp.int32),
            pltpu.VMEM((176,), jnp.int32),
            pltpu.VMEM((352,), jnp.int32),
            pltpu.VMEM((192,), jnp.int32),
            pltpu.SemaphoreType.DMA,
        ],
        compiler_params=pltpu.CompilerParams(needs_layout_passes=False),
    )(x, skey_pad, tzv, trv, rand)


def kernel(x, edge_index, aug_type):
    del aug_type
    gum_pad, rand_flat = _op_constants()
    edges = edge_index.reshape(2 * _E)
    hist = _sc_bincount(edges).reshape(_NW, _NPAD)
    mask_i, skey, tzv, trv = _tc_scores(hist, gum_pad)
    out = _sc_apply(x.reshape(_N * _D), skey.reshape(_NPAD),
                    tzv.reshape(16), trv.reshape(16), rand_flat)
    return out.reshape(_N, _D), mask_i.reshape(_N)


# speedup vs baseline: 1.0106x; 1.0106x over previous
import jax
import jax.numpy as jnp
from jax import lax
from jax.experimental import pallas as pl
from jax.experimental.pallas import tpu as pltpu
from jax.experimental.pallas import tpu_sc as plsc

_N = 10000
_E = 320000
_D = 128
_MASK_NUM = 1500
_REPL_NUM = 150
_NC = 2
_NS = 16
_NW = _NC * _NS
_EPW = 2 * _E // _NW
_RPW = 320
_LAST_BASE = _N - _RPW
_NPAD = _N + 16
_IMIN = jnp.iinfo(jnp.int32).min

def _op_constants():
    kg, kr = jax.random.split(jax.random.key(42))
    gum_pad = jnp.pad(
        jax.random.gumbel(kg, (_N,), dtype=jnp.float32),
        (0, _NPAD - _N)).reshape(1, _NPAD)
    rand_flat = jax.random.normal(
        kr, (_REPL_NUM, _D), dtype=jnp.float32).reshape(-1)
    return gum_pad, rand_flat


def _sget(ref, i):
    return ref[pl.ds(i, 16)][0]


def _bincount_body(edges_hbm, hist_hbm, ev, hv):
    c = lax.axis_index("c")
    s = lax.axis_index("s")
    wid = s * _NC + c
    base = wid * _EPW
    pltpu.sync_copy(edges_hbm.at[pl.ds(base, _EPW)], ev)
    z = jnp.zeros((16,), jnp.int32)

    @plsc.parallel_loop(0, _NPAD // 16, unroll=8)
    def _(i):
        hv[pl.ds(i * 16, 16)] = z

    ones = jnp.ones((16,), jnp.int32)

    @plsc.parallel_loop(0, _EPW // 16, unroll=8)
    def _(i):
        idx = ev[pl.ds(i * 16, 16)]
        plsc.addupdate_scatter(hv, [idx], ones)

    pltpu.sync_copy(hv, hist_hbm.at[wid, 0])


def _sc_bincount(edges):
    mesh = plsc.VectorSubcoreMesh(core_axis_name="c", subcore_axis_name="s")
    return pl.kernel(
        _bincount_body,
        out_type=jax.ShapeDtypeStruct((_NW, 1, _NPAD), jnp.int32),
        mesh=mesh,
        scratch_types=[
            pltpu.VMEM((_EPW,), jnp.int32),
            pltpu.VMEM((_NPAD,), jnp.int32),
        ],
        compiler_params=pltpu.CompilerParams(needs_layout_passes=False),
    )(edges)


def _tc_score_body(hist_ref, gum_ref, mask_ref, skey_ref, t_ref):
    col = lax.broadcasted_iota(jnp.int32, (1, _NPAD), 1)
    valid = col < _N
    deg = jnp.sum(hist_ref[...], axis=0, keepdims=True).astype(jnp.float32)
    m = jnp.max(deg)
    e = jnp.where(valid, jnp.exp(deg - m), jnp.float32(0.0))
    ssum = jnp.sum(e)
    prob = e / ssum
    sc = jnp.log(prob + jnp.float32(1e-20)) + gum_ref[...]
    b = lax.bitcast_convert_type(sc, jnp.int32)
    u = jnp.where(
        b < 0,
        lax.bitcast_convert_type(~b, jnp.uint32),
        lax.bitcast_convert_type(b, jnp.uint32) | jnp.uint32(0x80000000),
    )
    u = jnp.where(valid, u, jnp.uint32(0))

    def bit_body(t, ps):
        p1, p2 = ps
        sh = jnp.uint32(31) - t.astype(jnp.uint32)
        one = jnp.uint32(1) << sh
        c1 = p1 | one
        c2 = p2 | one
        n1 = jnp.sum((u >= c1).astype(jnp.int32))
        n2 = jnp.sum((u >= c2).astype(jnp.int32))
        return (jnp.where(n1 >= _MASK_NUM, c1, p1),
                jnp.where(n2 >= _REPL_NUM, c2, p2))

    tz, tr = lax.fori_loop(0, 32, bit_body, (jnp.uint32(0), jnp.uint32(0)))
    flip = jnp.uint32(0x80000000)
    mask_ref[...] = (u >= tz)[:, :_N]
    skey_ref[...] = lax.bitcast_convert_type(u ^ flip, jnp.int32)
    t_ref[...] = jnp.concatenate([
        jnp.full((1, 16), lax.bitcast_convert_type(tz ^ flip, jnp.int32),
                 jnp.int32),
        jnp.full((1, 16), lax.bitcast_convert_type(tr ^ flip, jnp.int32),
                 jnp.int32)], axis=1)


def _tc_scores(hist, gum2):
    return pl.pallas_call(
        _tc_score_body,
        out_shape=[
            jax.ShapeDtypeStruct((1, _N), jnp.bool_),
            jax.ShapeDtypeStruct((1, _NPAD), jnp.int32),
            jax.ShapeDtypeStruct((1, 32), jnp.int32),
        ],
    )(hist, gum2)


def _sc_apply_body(x_hbm, skey_hbm, t_hbm, rand_hbm, out_hbm,
                   buf, ub, tv, ckey, cidx, lidx, olist, semx):
    c = lax.axis_index("c")
    s = lax.axis_index("s")
    wid = s * _NC + c
    base = jnp.minimum(wid * _RPW, _LAST_BASE)
    cpx = pltpu.async_copy(x_hbm.at[pl.ds(base, _RPW)], buf, semx)
    pltpu.sync_copy(skey_hbm, ub)
    pltpu.sync_copy(t_hbm, tv)
    tz = tv[pl.ds(0, 16)]
    tr = tv[pl.ds(16, 16)]
    lane = lax.iota(jnp.int32, 16)
    imin = jnp.full((16,), _IMIN, jnp.int32)
    for q in range(11):
        ckey[pl.ds(q * 16, 16)] = imin

    def zscan(i, cnt):
        v = ub[pl.ds(base + i * 16, 16)]
        msk = v >= tz
        pc = plsc.all_reduce_population_count(msk)[0]

        @pl.when(pc > 0)
        def _():
            lv = i * 16 + lane
            plsc.store_compressed(lidx.at[pl.ds(cnt, 16)], lv, mask=msk)

        return cnt + pc

    zcnt = lax.fori_loop(0, _RPW // 16, zscan, jnp.int32(0))

    def cscan(i, cnt):
        v = ub[pl.ds(i * 16, 16)]
        msk = v >= tr
        pc = plsc.all_reduce_population_count(msk)[0]

        @pl.when(pc > 0)
        def _():
            gv = i * 16 + lane
            plsc.store_compressed(ckey.at[pl.ds(cnt, 16)], v, mask=msk)
            plsc.store_compressed(cidx.at[pl.ds(cnt, 16)], gv, mask=msk)

        return cnt + pc

    ccnt = lax.fori_loop(0, _NPAD // 16, cscan, jnp.int32(0))

    def oscan(q, cnt):
        jv = q * 16 + lane
        ci = cidx[pl.ds(q * 16, 16)]
        own = (jv < ccnt) & (ci >= base) & (ci < base + _RPW)
        plsc.store_compressed(olist.at[pl.ds(cnt, 16)], jv, mask=own)
        return cnt + plsc.all_reduce_population_count(own)[0]

    ocnt = lax.fori_loop(0, 11, oscan, jnp.int32(0))

    cpx.wait()
    zrow = jnp.zeros((16,), jnp.float32)

    def zero_body(j, _):
        r = _sget(lidx, j)
        for cc in range(8):
            buf[r, pl.ds(cc * 16, 16)] = zrow
        return 0

    lax.fori_loop(0, zcnt, zero_body, 0)

    def repl_body(jj, _):
        j = _sget(olist, jj)
        kj = _sget(ckey, j)
        gi = _sget(cidx, j)
        kjv = jnp.full((16,), kj, jnp.int32)
        giv = jnp.full((16,), gi, jnp.int32)

        def rk(q, r):
            ck = ckey[pl.ds(q * 16, 16)]
            ci = cidx[pl.ds(q * 16, 16)]
            cmp = (ck > kjv) | ((ck == kjv) & (ci < giv))
            return r + plsc.all_reduce_population_count(cmp)[0]

        rank = lax.fori_loop(0, 11, rk, jnp.int32(0))
        rank = jnp.minimum(rank, _REPL_NUM - 1)
        pltpu.sync_copy(rand_hbm.at[pl.ds(rank * _D, _D)],
                        buf.at[gi - base])
        return 0

    lax.fori_loop(0, ocnt, repl_body, 0)
    pltpu.sync_copy(buf, out_hbm.at[pl.ds(base, _RPW)])


def _sc_apply(x, skey_pad, tvals, rand):
    mesh = plsc.VectorSubcoreMesh(core_axis_name="c", subcore_axis_name="s")
    return pl.kernel(
        _sc_apply_body,
        out_type=jax.ShapeDtypeStruct((_N, _D), jnp.float32),
        mesh=mesh,
        scratch_types=[
            pltpu.VMEM((_RPW, _D), jnp.float32),
            pltpu.VMEM((_NPAD,), jnp.int32),
            pltpu.VMEM((32,), jnp.int32),
            pltpu.VMEM((176,), jnp.int32),
            pltpu.VMEM((176,), jnp.int32),
            pltpu.VMEM((352,), jnp.int32),
            pltpu.VMEM((192,), jnp.int32),
            pltpu.SemaphoreType.DMA,
        ],
        compiler_params=pltpu.CompilerParams(needs_layout_passes=False),
    )(x, skey_pad, tvals, rand)


def kernel(x, edge_index, aug_type):
    del aug_type
    gum_pad, rand_flat = _op_constants()
    edges = edge_index.reshape(2 * _E)
    hist = _sc_bincount(edges).reshape(_NW, _NPAD)
    mask_i, skey, tvals = _tc_scores(hist, gum_pad)
    out = _sc_apply(x, skey.reshape(_NPAD), tvals.reshape(32), rand_flat)
    return out, mask_i.reshape(_N)


# speedup vs baseline: 1.1656x; 1.1533x over previous
import jax
import jax.numpy as jnp
from jax import lax
from jax.experimental import pallas as pl
from jax.experimental.pallas import tpu as pltpu
from jax.experimental.pallas import tpu_sc as plsc

_N = 10000
_E = 320000
_D = 128
_MASK_NUM = 1500
_REPL_NUM = 150
_NC = 2
_NS = 16
_NW = _NC * _NS
_EPW = 2 * _E // _NW
_RPW = 320
_LAST_BASE = _N - _RPW
_NPAD = _N + 16
_IMIN = jnp.iinfo(jnp.int32).min

def _op_constants():
    def build():
        kg, kr = jax.random.split(jax.random.key(42))
        gum_pad = jnp.pad(
            jax.random.gumbel(kg, (_N,), dtype=jnp.float32),
            (0, _NPAD - _N)).reshape(1, _NPAD)
        rand_flat = jax.random.normal(
            kr, (_REPL_NUM, _D), dtype=jnp.float32).reshape(-1)
        return gum_pad, rand_flat

    try:
        with jax.ensure_compile_time_eval():
            return build()
    except Exception:
        return build()


def _sget(ref, i):
    return ref[pl.ds(i, 16)][0]


_ECHUNK = 10240
_ELAST = _E - _ECHUNK
_ECOV = 31 * _ECHUNK


def _bincount_body(edges_hbm, hist_hbm, ev, hv):
    c = lax.axis_index("c")
    s = lax.axis_index("s")
    wid = s * _NC + c
    base = jnp.minimum(wid * _ECHUNK, _ELAST)
    sv = (wid * _ECHUNK - base) // 16
    pltpu.sync_copy(edges_hbm.at[:, pl.ds(base, _ECHUNK)], ev)
    z = jnp.zeros((16,), jnp.int32)

    @plsc.parallel_loop(0, _NPAD // 16, unroll=8)
    def _(i):
        hv[pl.ds(i * 16, 16)] = z

    ones = jnp.ones((16,), jnp.int32)

    @plsc.parallel_loop(0, _ECHUNK // 16, unroll=8)
    def _(i):
        live = i >= sv
        for r in range(2):
            idx = ev[r, pl.ds(i * 16, 16)]
            plsc.addupdate_scatter(
                hv, [idx], ones, mask=jnp.full((16,), live, jnp.bool_))

    pltpu.sync_copy(hv, hist_hbm.at[wid, 0])


def _sc_bincount(edges):
    mesh = plsc.VectorSubcoreMesh(core_axis_name="c", subcore_axis_name="s")
    return pl.kernel(
        _bincount_body,
        out_type=jax.ShapeDtypeStruct((_NW, 1, _NPAD), jnp.int32),
        mesh=mesh,
        scratch_types=[
            pltpu.VMEM((2, _ECHUNK), jnp.int32),
            pltpu.VMEM((_NPAD,), jnp.int32),
        ],
        compiler_params=pltpu.CompilerParams(needs_layout_passes=False),
    )(edges)


def _tc_score_body(hist_ref, gum_ref, mask_ref, skey_ref, t_ref):
    col = lax.broadcasted_iota(jnp.int32, (1, _NPAD), 1)
    valid = col < _N
    deg = jnp.sum(hist_ref[...][:, 0, :], axis=0,
                  keepdims=True).astype(jnp.float32)
    m = jnp.max(deg)
    e = jnp.where(valid, jnp.exp(deg - m), jnp.float32(0.0))
    ssum = jnp.sum(e)
    prob = e / ssum
    sc = jnp.log(prob + jnp.float32(1e-20)) + gum_ref[...]
    b = lax.bitcast_convert_type(sc, jnp.int32)
    u = jnp.where(
        b < 0,
        lax.bitcast_convert_type(~b, jnp.uint32),
        lax.bitcast_convert_type(b, jnp.uint32) | jnp.uint32(0x80000000),
    )
    u = jnp.where(valid, u, jnp.uint32(0))

    def bit_body(t, ps):
        p1, p2 = ps
        sh = jnp.uint32(31) - t.astype(jnp.uint32)
        one = jnp.uint32(1) << sh
        c1 = p1 | one
        c2 = p2 | one
        n1 = jnp.sum((u >= c1).astype(jnp.int32))
        n2 = jnp.sum((u >= c2).astype(jnp.int32))
        return (jnp.where(n1 >= _MASK_NUM, c1, p1),
                jnp.where(n2 >= _REPL_NUM, c2, p2))

    tz, tr = lax.fori_loop(0, 32, bit_body, (jnp.uint32(0), jnp.uint32(0)))
    flip = jnp.uint32(0x80000000)
    mask_ref[...] = (u >= tz)[0, :_N]
    skey_ref[...] = lax.bitcast_convert_type(u ^ flip, jnp.int32)[0]
    t_ref[...] = jnp.concatenate([
        jnp.full((16,), lax.bitcast_convert_type(tz ^ flip, jnp.int32),
                 jnp.int32),
        jnp.full((16,), lax.bitcast_convert_type(tr ^ flip, jnp.int32),
                 jnp.int32)])


def _tc_scores(hist, gum2):
    return pl.pallas_call(
        _tc_score_body,
        out_shape=[
            jax.ShapeDtypeStruct((_N,), jnp.bool_),
            jax.ShapeDtypeStruct((_NPAD,), jnp.int32),
            jax.ShapeDtypeStruct((32,), jnp.int32),
        ],
    )(hist, gum2)


def _sc_apply_body(x_hbm, skey_hbm, t_hbm, rand_hbm, out_hbm,
                   buf, ub, tv, ckey, cidx, lidx, olist, semx):
    c = lax.axis_index("c")
    s = lax.axis_index("s")
    wid = s * _NC + c
    base = jnp.minimum(wid * _RPW, _LAST_BASE)
    cpx = pltpu.async_copy(x_hbm.at[pl.ds(base, _RPW)], buf, semx)
    pltpu.sync_copy(skey_hbm, ub)
    pltpu.sync_copy(t_hbm, tv)
    tz = tv[pl.ds(0, 16)]
    tr = tv[pl.ds(16, 16)]
    lane = lax.iota(jnp.int32, 16)
    imin = jnp.full((16,), _IMIN, jnp.int32)
    for q in range(11):
        ckey[pl.ds(q * 16, 16)] = imin

    def zscan(i, cnt):
        v = ub[pl.ds(base + i * 16, 16)]
        msk = v >= tz
        pc = plsc.all_reduce_population_count(msk)[0]

        @pl.when(pc > 0)
        def _():
            lv = i * 16 + lane
            plsc.store_compressed(lidx.at[pl.ds(cnt, 16)], lv, mask=msk)

        return cnt + pc

    zcnt = lax.fori_loop(0, _RPW // 16, zscan, jnp.int32(0))

    def cscan(i, cnt):
        v = ub[pl.ds(i * 16, 16)]
        msk = v >= tr
        pc = plsc.all_reduce_population_count(msk)[0]

        @pl.when(pc > 0)
        def _():
            gv = i * 16 + lane
            plsc.store_compressed(ckey.at[pl.ds(cnt, 16)], v, mask=msk)
            plsc.store_compressed(cidx.at[pl.ds(cnt, 16)], gv, mask=msk)

        return cnt + pc

    ccnt = lax.fori_loop(0, _NPAD // 16, cscan, jnp.int32(0))

    def oscan(q, cnt):
        jv = q * 16 + lane
        ci = cidx[pl.ds(q * 16, 16)]
        own = (jv < ccnt) & (ci >= base) & (ci < base + _RPW)
        plsc.store_compressed(olist.at[pl.ds(cnt, 16)], jv, mask=own)
        return cnt + plsc.all_reduce_population_count(own)[0]

    ocnt = lax.fori_loop(0, 11, oscan, jnp.int32(0))

    cpx.wait()
    zrow = jnp.zeros((16,), jnp.float32)

    def zero_body(j, _):
        r = _sget(lidx, j)
        for cc in range(8):
            buf[r, pl.ds(cc * 16, 16)] = zrow
        return 0

    lax.fori_loop(0, zcnt, zero_body, 0)

    def repl_body(jj, _):
        j = _sget(olist, jj)
        kj = _sget(ckey, j)
        gi = _sget(cidx, j)
        kjv = jnp.full((16,), kj, jnp.int32)
        giv = jnp.full((16,), gi, jnp.int32)

        def rk(q, r):
            ck = ckey[pl.ds(q * 16, 16)]
            ci = cidx[pl.ds(q * 16, 16)]
            cmp = (ck > kjv) | ((ck == kjv) & (ci < giv))
            return r + plsc.all_reduce_population_count(cmp)[0]

        rank = lax.fori_loop(0, 11, rk, jnp.int32(0))
        rank = jnp.minimum(rank, _REPL_NUM - 1)
        pltpu.sync_copy(rand_hbm.at[pl.ds(rank * _D, _D)],
                        buf.at[gi - base])
        return 0

    lax.fori_loop(0, ocnt, repl_body, 0)
    pltpu.sync_copy(buf, out_hbm.at[pl.ds(base, _RPW)])


def _sc_apply(x, skey_pad, tvals, rand):
    mesh = plsc.VectorSubcoreMesh(core_axis_name="c", subcore_axis_name="s")
    return pl.kernel(
        _sc_apply_body,
        out_type=jax.ShapeDtypeStruct((_N, _D), jnp.float32),
        mesh=mesh,
        scratch_types=[
            pltpu.VMEM((_RPW, _D), jnp.float32),
            pltpu.VMEM((_NPAD,), jnp.int32),
            pltpu.VMEM((32,), jnp.int32),
            pltpu.VMEM((176,), jnp.int32),
            pltpu.VMEM((176,), jnp.int32),
            pltpu.VMEM((352,), jnp.int32),
            pltpu.VMEM((192,), jnp.int32),
            pltpu.SemaphoreType.DMA,
        ],
        compiler_params=pltpu.CompilerParams(needs_layout_passes=False),
    )(x, skey_pad, tvals, rand)


def kernel(x, edge_index, aug_type):
    del aug_type
    gum_pad, rand_flat = _op_constants()
    hist = _sc_bincount(edge_index)
    mask_i, skey, tvals = _tc_scores(hist, gum_pad)
    out = _sc_apply(x, skey, tvals, rand_flat)
    return out, mask_i


# speedup vs baseline: 1.2453x; 1.0684x over previous
import jax
import jax.numpy as jnp
from jax import lax
from jax.experimental import pallas as pl
from jax.experimental.pallas import tpu as pltpu
from jax.experimental.pallas import tpu_sc as plsc

_N = 10000
_E = 320000
_D = 128
_MASK_NUM = 1500
_REPL_NUM = 150
_NC = 2
_NS = 16
_NW = _NC * _NS
_EPW = 2 * _E // _NW
_RPW = 320
_LAST_BASE = _N - _RPW
_NPAD = _N + 16
_IMIN = jnp.iinfo(jnp.int32).min

def _op_constants():
    def build():
        kg, kr = jax.random.split(jax.random.key(42))
        gum_pad = jnp.pad(
            jax.random.gumbel(kg, (_N,), dtype=jnp.float32),
            (0, _NPAD - _N)).reshape(1, _NPAD)
        rand_flat = jax.random.normal(
            kr, (_REPL_NUM, _D), dtype=jnp.float32).reshape(-1)
        return gum_pad, rand_flat

    try:
        with jax.ensure_compile_time_eval():
            return build()
    except Exception:
        return build()


def _sget(ref, i):
    return ref[pl.ds(i, 16)][0]


_ECHUNK = 10240
_ELAST = _E - _ECHUNK
_ECOV = 31 * _ECHUNK


def _bincount_body(edges_hbm, hist_hbm, ev, hv):
    c = lax.axis_index("c")
    s = lax.axis_index("s")
    wid = s * _NC + c
    base = jnp.minimum(wid * _ECHUNK, _ELAST)
    sv = (wid * _ECHUNK - base) // 16
    pltpu.sync_copy(edges_hbm.at[:, pl.ds(base, _ECHUNK)], ev)
    z = jnp.zeros((16,), jnp.int32)

    @plsc.parallel_loop(0, _NPAD // 16, unroll=8)
    def _(i):
        hv[pl.ds(i * 16, 16)] = z

    ones = jnp.ones((16,), jnp.int32)

    @plsc.parallel_loop(0, _ECHUNK // 16, unroll=8)
    def _(i):
        live = i >= sv
        for r in range(2):
            idx = ev[r, pl.ds(i * 16, 16)]
            plsc.addupdate_scatter(
                hv, [idx], ones, mask=jnp.full((16,), live, jnp.bool_))

    pltpu.sync_copy(hv, hist_hbm.at[wid, 0])


def _sc_bincount(edges):
    mesh = plsc.VectorSubcoreMesh(core_axis_name="c", subcore_axis_name="s")
    return pl.kernel(
        _bincount_body,
        out_type=jax.ShapeDtypeStruct((_NW, 1, _NPAD), jnp.int32),
        mesh=mesh,
        scratch_types=[
            pltpu.VMEM((2, _ECHUNK), jnp.int32),
            pltpu.VMEM((_NPAD,), jnp.int32),
        ],
        compiler_params=pltpu.CompilerParams(needs_layout_passes=False),
    )(edges)


def _tc_score_body(hist_ref, gum_ref, mask_ref, skey_ref, t_ref):
    col = lax.broadcasted_iota(jnp.int32, (1, _NPAD), 1)
    valid = col < _N
    deg = jnp.sum(hist_ref[...][:, 0, :], axis=0,
                  keepdims=True).astype(jnp.float32)
    m = jnp.max(deg)
    e = jnp.where(valid, jnp.exp(deg - m), jnp.float32(0.0))
    ssum = jnp.sum(e)
    prob = e / ssum
    sc = jnp.log(prob + jnp.float32(1e-20)) + gum_ref[...]
    b = lax.bitcast_convert_type(sc, jnp.int32)
    u = jnp.where(
        b < 0,
        lax.bitcast_convert_type(~b, jnp.uint32),
        lax.bitcast_convert_type(b, jnp.uint32) | jnp.uint32(0x80000000),
    )
    u = jnp.where(valid, u, jnp.uint32(0))

    def bit_body(t, ps):
        p1, p2 = ps
        sh = jnp.uint32(31) - t.astype(jnp.uint32)
        one = jnp.uint32(1) << sh
        c1 = p1 | one
        c2 = p2 | one
        n1 = jnp.sum((u >= c1).astype(jnp.int32))
        n2 = jnp.sum((u >= c2).astype(jnp.int32))
        return (jnp.where(n1 >= _MASK_NUM, c1, p1),
                jnp.where(n2 >= _REPL_NUM, c2, p2))

    tz, tr = lax.fori_loop(0, 32, bit_body, (jnp.uint32(0), jnp.uint32(0)))
    flip = jnp.uint32(0x80000000)
    mask_ref[...] = (u >= tz)[0, :_N]
    skey_ref[...] = lax.bitcast_convert_type(u ^ flip, jnp.int32)[0]
    t_ref[...] = jnp.concatenate([
        jnp.full((16,), lax.bitcast_convert_type(tz ^ flip, jnp.int32),
                 jnp.int32),
        jnp.full((16,), lax.bitcast_convert_type(tr ^ flip, jnp.int32),
                 jnp.int32)])


def _tc_scores(hist, gum2):
    return pl.pallas_call(
        _tc_score_body,
        out_shape=[
            jax.ShapeDtypeStruct((_N,), jnp.bool_),
            jax.ShapeDtypeStruct((_NPAD,), jnp.int32),
            jax.ShapeDtypeStruct((32,), jnp.int32),
        ],
    )(hist, gum2)


def _sc_apply_body(x_hbm, skey_hbm, t_hbm, rand_hbm, out_hbm,
                   buf, ub, tv, ckey, cidx, lidx, olist, semx):
    c = lax.axis_index("c")
    s = lax.axis_index("s")
    wid = s * _NC + c
    base = jnp.minimum(wid * _RPW, _LAST_BASE)
    cpx = pltpu.async_copy(x_hbm.at[pl.ds(base, _RPW)], buf, semx)
    pltpu.sync_copy(skey_hbm, ub)
    pltpu.sync_copy(t_hbm, tv)
    tz = tv[pl.ds(0, 16)]
    tr = tv[pl.ds(16, 16)]
    lane = lax.iota(jnp.int32, 16)
    imin = jnp.full((16,), _IMIN, jnp.int32)
    for q in range(11):
        ckey[pl.ds(q * 16, 16)] = imin

    def zscan(i, cnt):
        v = ub[pl.ds(base + i * 16, 16)]
        msk = v >= tz
        pc = plsc.all_reduce_population_count(msk)[0]

        @pl.when(pc > 0)
        def _():
            lv = i * 16 + lane
            plsc.store_compressed(lidx.at[pl.ds(cnt, 16)], lv, mask=msk)

        return cnt + pc

    zcnt = lax.fori_loop(0, _RPW // 16, zscan, jnp.int32(0))

    def cscan(i, cnt):
        v0 = ub[pl.ds(i * 32, 16)]
        v1 = ub[pl.ds(i * 32 + 16, 16)]
        m0 = v0 >= tr
        m1 = v1 >= tr
        pc0 = plsc.all_reduce_population_count(m0)[0]
        pc1 = plsc.all_reduce_population_count(m1)[0]

        @pl.when(pc0 + pc1 > 0)
        def _():
            gv = i * 32 + lane
            plsc.store_compressed(ckey.at[pl.ds(cnt, 16)], v0, mask=m0)
            plsc.store_compressed(cidx.at[pl.ds(cnt, 16)], gv, mask=m0)
            plsc.store_compressed(ckey.at[pl.ds(cnt + pc0, 16)], v1, mask=m1)
            plsc.store_compressed(cidx.at[pl.ds(cnt + pc0, 16)], gv + 16,
                                  mask=m1)

        return cnt + pc0 + pc1

    ccnt = lax.fori_loop(0, _NPAD // 32, cscan, jnp.int32(0), unroll=2)

    def oscan(q, cnt):
        jv = q * 16 + lane
        ci = cidx[pl.ds(q * 16, 16)]
        own = (jv < ccnt) & (ci >= base) & (ci < base + _RPW)
        plsc.store_compressed(olist.at[pl.ds(cnt, 16)], jv, mask=own)
        return cnt + plsc.all_reduce_population_count(own)[0]

    ocnt = lax.fori_loop(0, 11, oscan, jnp.int32(0))

    cpx.wait()
    zrow = jnp.zeros((16,), jnp.float32)

    def zero_body(j, _):
        r = _sget(lidx, j)
        for cc in range(8):
            buf[r, pl.ds(cc * 16, 16)] = zrow
        return 0

    lax.fori_loop(0, zcnt, zero_body, 0)

    def repl_body(jj, _):
        j = _sget(olist, jj)
        kj = _sget(ckey, j)
        gi = _sget(cidx, j)
        kjv = jnp.full((16,), kj, jnp.int32)
        giv = jnp.full((16,), gi, jnp.int32)

        def rk(q, r):
            ck = ckey[pl.ds(q * 16, 16)]
            ci = cidx[pl.ds(q * 16, 16)]
            cmp = (ck > kjv) | ((ck == kjv) & (ci < giv))
            return r + plsc.all_reduce_population_count(cmp)[0]

        rank = lax.fori_loop(0, 11, rk, jnp.int32(0))
        rank = jnp.minimum(rank, _REPL_NUM - 1)
        pltpu.sync_copy(rand_hbm.at[pl.ds(rank * _D, _D)],
                        buf.at[gi - base])
        return 0

    lax.fori_loop(0, ocnt, repl_body, 0)
    pltpu.sync_copy(buf, out_hbm.at[pl.ds(base, _RPW)])


def _sc_apply(x, skey_pad, tvals, rand):
    mesh = plsc.VectorSubcoreMesh(core_axis_name="c", subcore_axis_name="s")
    return pl.kernel(
        _sc_apply_body,
        out_type=jax.ShapeDtypeStruct((_N, _D), jnp.float32),
        mesh=mesh,
        scratch_types=[
            pltpu.VMEM((_RPW, _D), jnp.float32),
            pltpu.VMEM((_NPAD,), jnp.int32),
            pltpu.VMEM((32,), jnp.int32),
            pltpu.VMEM((176,), jnp.int32),
            pltpu.VMEM((176,), jnp.int32),
            pltpu.VMEM((352,), jnp.int32),
            pltpu.VMEM((192,), jnp.int32),
            pltpu.SemaphoreType.DMA,
        ],
        compiler_params=pltpu.CompilerParams(needs_layout_passes=False),
    )(x, skey_pad, tvals, rand)


def kernel(x, edge_index, aug_type):
    del aug_type
    gum_pad, rand_flat = _op_constants()
    hist = _sc_bincount(edge_index)
    mask_i, skey, tvals = _tc_scores(hist, gum_pad)
    out = _sc_apply(x, skey, tvals, rand_flat)
    return out, mask_i


# speedup vs baseline: 1.2878x; 1.0341x over previous
import jax
import jax.numpy as jnp
from jax import lax
from jax.experimental import pallas as pl
from jax.experimental.pallas import tpu as pltpu
from jax.experimental.pallas import tpu_sc as plsc

_N = 10000
_E = 320000
_D = 128
_MASK_NUM = 1500
_REPL_NUM = 150
_NC = 2
_NS = 16
_NW = _NC * _NS
_EPW = 2 * _E // _NW
_RPW = 320
_LAST_BASE = _N - _RPW
_NPAD = _N + 48
_IMIN = jnp.iinfo(jnp.int32).min

def _op_constants():
    def build():
        kg, kr = jax.random.split(jax.random.key(42))
        gum_pad = jnp.pad(
            jax.random.gumbel(kg, (_N,), dtype=jnp.float32),
            (0, _NPAD - _N)).reshape(1, _NPAD)
        rand_flat = jax.random.normal(
            kr, (_REPL_NUM, _D), dtype=jnp.float32).reshape(-1)
        return gum_pad, rand_flat

    try:
        with jax.ensure_compile_time_eval():
            return build()
    except Exception:
        return build()


def _sget(ref, i):
    return ref[pl.ds(i, 16)][0]


_ECHUNK = 10240
_ELAST = _E - _ECHUNK
_ECOV = 31 * _ECHUNK


def _bincount_body(edges_hbm, hist_hbm, ev, hv):
    c = lax.axis_index("c")
    s = lax.axis_index("s")
    wid = s * _NC + c
    base = jnp.minimum(wid * _ECHUNK, _ELAST)
    sv = (wid * _ECHUNK - base) // 16
    pltpu.sync_copy(edges_hbm.at[:, pl.ds(base, _ECHUNK)], ev)
    z = jnp.zeros((16,), jnp.int32)

    @plsc.parallel_loop(0, _NPAD // 16, unroll=8)
    def _(i):
        hv[pl.ds(i * 16, 16)] = z

    ones = jnp.ones((16,), jnp.int32)

    @plsc.parallel_loop(0, _ECHUNK // 16, unroll=8)
    def _(i):
        live = i >= sv
        for r in range(2):
            idx = ev[r, pl.ds(i * 16, 16)]
            plsc.addupdate_scatter(
                hv, [idx], ones, mask=jnp.full((16,), live, jnp.bool_))

    pltpu.sync_copy(hv, hist_hbm.at[wid, 0])


def _sc_bincount(edges):
    mesh = plsc.VectorSubcoreMesh(core_axis_name="c", subcore_axis_name="s")
    return pl.kernel(
        _bincount_body,
        out_type=jax.ShapeDtypeStruct((_NW, 1, _NPAD), jnp.int32),
        mesh=mesh,
        scratch_types=[
            pltpu.VMEM((2, _ECHUNK), jnp.int32),
            pltpu.VMEM((_NPAD,), jnp.int32),
        ],
        compiler_params=pltpu.CompilerParams(needs_layout_passes=False),
    )(edges)


def _tc_score_body(hist_ref, gum_ref, mask_ref, skey_ref, t_ref):
    col = lax.broadcasted_iota(jnp.int32, (1, _NPAD), 1)
    valid = col < _N
    deg = jnp.sum(hist_ref[...][:, 0, :], axis=0,
                  keepdims=True).astype(jnp.float32)
    m = jnp.max(deg)
    e = jnp.where(valid, jnp.exp(deg - m), jnp.float32(0.0))
    ssum = jnp.sum(e)
    prob = e / ssum
    sc = jnp.log(prob + jnp.float32(1e-20)) + gum_ref[...]
    b = lax.bitcast_convert_type(sc, jnp.int32)
    u = jnp.where(
        b < 0,
        lax.bitcast_convert_type(~b, jnp.uint32),
        lax.bitcast_convert_type(b, jnp.uint32) | jnp.uint32(0x80000000),
    )
    u = jnp.where(valid, u, jnp.uint32(0))

    def bit_body(t, ps):
        p1, p2 = ps
        sh = jnp.uint32(31) - t.astype(jnp.uint32)
        one = jnp.uint32(1) << sh
        c1 = p1 | one
        c2 = p2 | one
        n1 = jnp.sum((u >= c1).astype(jnp.int32))
        n2 = jnp.sum((u >= c2).astype(jnp.int32))
        return (jnp.where(n1 >= _MASK_NUM, c1, p1),
                jnp.where(n2 >= _REPL_NUM, c2, p2))

    tz, tr = lax.fori_loop(0, 32, bit_body, (jnp.uint32(0), jnp.uint32(0)))
    flip = jnp.uint32(0x80000000)
    mask_ref[...] = (u >= tz)[0, :_N]
    skey_ref[...] = lax.bitcast_convert_type(u ^ flip, jnp.int32)[0]
    t_ref[...] = jnp.concatenate([
        jnp.full((16,), lax.bitcast_convert_type(tz ^ flip, jnp.int32),
                 jnp.int32),
        jnp.full((16,), lax.bitcast_convert_type(tr ^ flip, jnp.int32),
                 jnp.int32)])


def _tc_scores(hist, gum2):
    return pl.pallas_call(
        _tc_score_body,
        out_shape=[
            jax.ShapeDtypeStruct((_N,), jnp.bool_),
            jax.ShapeDtypeStruct((_NPAD,), jnp.int32),
            jax.ShapeDtypeStruct((32,), jnp.int32),
        ],
    )(hist, gum2)


def _sc_apply_body(x_hbm, skey_hbm, t_hbm, rand_hbm, out_hbm,
                   buf, ub, tv, ckey, cidx, lidx, olist, semx):
    c = lax.axis_index("c")
    s = lax.axis_index("s")
    wid = s * _NC + c
    base = jnp.minimum(wid * _RPW, _LAST_BASE)
    cpx = pltpu.async_copy(x_hbm.at[pl.ds(base, _RPW)], buf, semx)
    pltpu.sync_copy(skey_hbm, ub)
    pltpu.sync_copy(t_hbm, tv)
    tz = tv[pl.ds(0, 16)]
    tr = tv[pl.ds(16, 16)]
    lane = lax.iota(jnp.int32, 16)
    imin = jnp.full((16,), _IMIN, jnp.int32)
    for q in range(11):
        ckey[pl.ds(q * 16, 16)] = imin

    def zscan(i, cnt):
        v0 = ub[pl.ds(base + i * 32, 16)]
        v1 = ub[pl.ds(base + i * 32 + 16, 16)]
        m0 = v0 >= tz
        m1 = v1 >= tz
        pc0 = plsc.all_reduce_population_count(m0)[0]
        pc1 = plsc.all_reduce_population_count(m1)[0]

        @pl.when(pc0 + pc1 > 0)
        def _():
            lv = i * 32 + lane
            plsc.store_compressed(lidx.at[pl.ds(cnt, 16)], lv, mask=m0)
            plsc.store_compressed(lidx.at[pl.ds(cnt + pc0, 16)], lv + 16,
                                  mask=m1)

        return cnt + pc0 + pc1

    zcnt = lax.fori_loop(0, _RPW // 32, zscan, jnp.int32(0))

    def cscan(i, cnt):
        vs = [ub[pl.ds(i * 64 + 16 * h, 16)] for h in range(4)]
        ms = [v >= tr for v in vs]
        pcs = [plsc.all_reduce_population_count(m)[0] for m in ms]
        tot = pcs[0] + pcs[1] + pcs[2] + pcs[3]

        @pl.when(tot > 0)
        def _():
            gv = i * 64 + lane
            off = cnt
            for h in range(4):
                plsc.store_compressed(ckey.at[pl.ds(off, 16)], vs[h],
                                      mask=ms[h])
                plsc.store_compressed(cidx.at[pl.ds(off, 16)], gv + 16 * h,
                                      mask=ms[h])
                off = off + pcs[h]

        return cnt + tot

    ccnt = lax.fori_loop(0, _NPAD // 64, cscan, jnp.int32(0), unroll=2)

    def oscan(q, cnt):
        jv = q * 16 + lane
        ci = cidx[pl.ds(q * 16, 16)]
        own = (jv < ccnt) & (ci >= base) & (ci < base + _RPW)
        plsc.store_compressed(olist.at[pl.ds(cnt, 16)], jv, mask=own)
        return cnt + plsc.all_reduce_population_count(own)[0]

    ocnt = lax.fori_loop(0, 11, oscan, jnp.int32(0))

    cpx.wait()
    zrow = jnp.zeros((16,), jnp.float32)

    def zero_body(j, _):
        r = _sget(lidx, j)
        for cc in range(8):
            buf[r, pl.ds(cc * 16, 16)] = zrow
        return 0

    lax.fori_loop(0, zcnt, zero_body, 0)

    def repl_body(jj, _):
        j = _sget(olist, jj)
        kj = _sget(ckey, j)
        gi = _sget(cidx, j)
        kjv = jnp.full((16,), kj, jnp.int32)
        giv = jnp.full((16,), gi, jnp.int32)

        def rk(q, r):
            ck = ckey[pl.ds(q * 16, 16)]
            ci = cidx[pl.ds(q * 16, 16)]
            cmp = (ck > kjv) | ((ck == kjv) & (ci < giv))
            return r + plsc.all_reduce_population_count(cmp)[0]

        rank = lax.fori_loop(0, 11, rk, jnp.int32(0))
        rank = jnp.minimum(rank, _REPL_NUM - 1)
        pltpu.sync_copy(rand_hbm.at[pl.ds(rank * _D, _D)],
                        buf.at[gi - base])
        return 0

    lax.fori_loop(0, ocnt, repl_body, 0)
    pltpu.sync_copy(buf, out_hbm.at[pl.ds(base, _RPW)])


def _sc_apply(x, skey_pad, tvals, rand):
    mesh = plsc.VectorSubcoreMesh(core_axis_name="c", subcore_axis_name="s")
    return pl.kernel(
        _sc_apply_body,
        out_type=jax.ShapeDtypeStruct((_N, _D), jnp.float32),
        mesh=mesh,
        scratch_types=[
            pltpu.VMEM((_RPW, _D), jnp.float32),
            pltpu.VMEM((_NPAD,), jnp.int32),
            pltpu.VMEM((32,), jnp.int32),
            pltpu.VMEM((176,), jnp.int32),
            pltpu.VMEM((176,), jnp.int32),
            pltpu.VMEM((352,), jnp.int32),
            pltpu.VMEM((192,), jnp.int32),
            pltpu.SemaphoreType.DMA,
        ],
        compiler_params=pltpu.CompilerParams(needs_layout_passes=False),
    )(x, skey_pad, tvals, rand)


def kernel(x, edge_index, aug_type):
    del aug_type
    gum_pad, rand_flat = _op_constants()
    hist = _sc_bincount(edge_index)
    mask_i, skey, tvals = _tc_scores(hist, gum_pad)
    out = _sc_apply(x, skey, tvals, rand_flat)
    return out, mask_i


# speedup vs baseline: 1.3036x; 1.0123x over previous
import jax
import jax.numpy as jnp
from jax import lax
from jax.experimental import pallas as pl
from jax.experimental.pallas import tpu as pltpu
from jax.experimental.pallas import tpu_sc as plsc

_N = 10000
_E = 320000
_D = 128
_MASK_NUM = 1500
_REPL_NUM = 150
_NC = 2
_NS = 16
_NW = _NC * _NS
_EPW = 2 * _E // _NW
_RPW = 320
_LAST_BASE = _N - _RPW
_NPAD = _N + 112
_IMIN = jnp.iinfo(jnp.int32).min

def _op_constants():
    def build():
        kg, kr = jax.random.split(jax.random.key(42))
        gum_pad = jnp.pad(
            jax.random.gumbel(kg, (_N,), dtype=jnp.float32),
            (0, _NPAD - _N)).reshape(1, _NPAD)
        rand_flat = jax.random.normal(
            kr, (_REPL_NUM, _D), dtype=jnp.float32).reshape(-1)
        return gum_pad, rand_flat

    try:
        with jax.ensure_compile_time_eval():
            return build()
    except Exception:
        return build()


def _sget(ref, i):
    return ref[pl.ds(i, 16)][0]


_ECHUNK = 10240
_ELAST = _E - _ECHUNK
_ECOV = 31 * _ECHUNK


def _bincount_body(edges_hbm, hist_hbm, ev, hv):
    c = lax.axis_index("c")
    s = lax.axis_index("s")
    wid = s * _NC + c
    base = jnp.minimum(wid * _ECHUNK, _ELAST)
    sv = (wid * _ECHUNK - base) // 16
    pltpu.sync_copy(edges_hbm.at[:, pl.ds(base, _ECHUNK)], ev)
    z = jnp.zeros((16,), jnp.int32)

    @plsc.parallel_loop(0, _NPAD // 16, unroll=8)
    def _(i):
        hv[pl.ds(i * 16, 16)] = z

    ones = jnp.ones((16,), jnp.int32)

    @plsc.parallel_loop(0, _ECHUNK // 16, unroll=8)
    def _(i):
        live = i >= sv
        for r in range(2):
            idx = ev[r, pl.ds(i * 16, 16)]
            plsc.addupdate_scatter(
                hv, [idx], ones, mask=jnp.full((16,), live, jnp.bool_))

    pltpu.sync_copy(hv, hist_hbm.at[wid, 0])


def _sc_bincount(edges):
    mesh = plsc.VectorSubcoreMesh(core_axis_name="c", subcore_axis_name="s")
    return pl.kernel(
        _bincount_body,
        out_type=jax.ShapeDtypeStruct((_NW, 1, _NPAD), jnp.int32),
        mesh=mesh,
        scratch_types=[
            pltpu.VMEM((2, _ECHUNK), jnp.int32),
            pltpu.VMEM((_NPAD,), jnp.int32),
        ],
        compiler_params=pltpu.CompilerParams(needs_layout_passes=False),
    )(edges)


def _tc_score_body(hist_ref, gum_ref, mask_ref, skey_ref, t_ref):
    col = lax.broadcasted_iota(jnp.int32, (1, _NPAD), 1)
    valid = col < _N
    deg = jnp.sum(hist_ref[...][:, 0, :], axis=0,
                  keepdims=True).astype(jnp.float32)
    m = jnp.max(deg)
    e = jnp.where(valid, jnp.exp(deg - m), jnp.float32(0.0))
    ssum = jnp.sum(e)
    prob = e / ssum
    sc = jnp.log(prob + jnp.float32(1e-20)) + gum_ref[...]
    b = lax.bitcast_convert_type(sc, jnp.int32)
    u = jnp.where(
        b < 0,
        lax.bitcast_convert_type(~b, jnp.uint32),
        lax.bitcast_convert_type(b, jnp.uint32) | jnp.uint32(0x80000000),
    )
    u = jnp.where(valid, u, jnp.uint32(0))

    def bit_body(t, ps):
        p1, p2 = ps
        sh = jnp.uint32(31) - t.astype(jnp.uint32)
        one = jnp.uint32(1) << sh
        c1 = p1 | one
        c2 = p2 | one
        n1 = jnp.sum((u >= c1).astype(jnp.int32))
        n2 = jnp.sum((u >= c2).astype(jnp.int32))
        return (jnp.where(n1 >= _MASK_NUM, c1, p1),
                jnp.where(n2 >= _REPL_NUM, c2, p2))

    tz, tr = lax.fori_loop(0, 32, bit_body, (jnp.uint32(0), jnp.uint32(0)))
    flip = jnp.uint32(0x80000000)
    mask_ref[...] = (u >= tz)[0, :_N]
    skey_ref[...] = lax.bitcast_convert_type(u ^ flip, jnp.int32)[0]
    t_ref[...] = jnp.concatenate([
        jnp.full((16,), lax.bitcast_convert_type(tz ^ flip, jnp.int32),
                 jnp.int32),
        jnp.full((16,), lax.bitcast_convert_type(tr ^ flip, jnp.int32),
                 jnp.int32)])


def _tc_scores(hist, gum2):
    return pl.pallas_call(
        _tc_score_body,
        out_shape=[
            jax.ShapeDtypeStruct((_N,), jnp.bool_),
            jax.ShapeDtypeStruct((_NPAD,), jnp.int32),
            jax.ShapeDtypeStruct((32,), jnp.int32),
        ],
    )(hist, gum2)


def _sc_apply_body(x_hbm, skey_hbm, t_hbm, rand_hbm, out_hbm,
                   buf, ub, tv, ckey, cidx, lidx, olist, semx):
    c = lax.axis_index("c")
    s = lax.axis_index("s")
    wid = s * _NC + c
    base = jnp.minimum(wid * _RPW, _LAST_BASE)
    cpx = pltpu.async_copy(x_hbm.at[pl.ds(base, _RPW)], buf, semx)
    pltpu.sync_copy(skey_hbm, ub)
    pltpu.sync_copy(t_hbm, tv)
    tz = tv[pl.ds(0, 16)]
    tr = tv[pl.ds(16, 16)]
    lane = lax.iota(jnp.int32, 16)
    imin = jnp.full((16,), _IMIN, jnp.int32)
    for q in range(11):
        ckey[pl.ds(q * 16, 16)] = imin

    def zscan(i, cnt):
        v0 = ub[pl.ds(base + i * 32, 16)]
        v1 = ub[pl.ds(base + i * 32 + 16, 16)]
        m0 = v0 >= tz
        m1 = v1 >= tz
        pc0 = plsc.all_reduce_population_count(m0)[0]
        pc1 = plsc.all_reduce_population_count(m1)[0]

        @pl.when(pc0 + pc1 > 0)
        def _():
            lv = i * 32 + lane
            plsc.store_compressed(lidx.at[pl.ds(cnt, 16)], lv, mask=m0)
            plsc.store_compressed(lidx.at[pl.ds(cnt + pc0, 16)], lv + 16,
                                  mask=m1)

        return cnt + pc0 + pc1

    zcnt = lax.fori_loop(0, _RPW // 32, zscan, jnp.int32(0))

    def cscan(i, cnt):
        vs = [ub[pl.ds(i * 128 + 16 * h, 16)] for h in range(8)]
        ms = [v >= tr for v in vs]
        pcs = [plsc.all_reduce_population_count(m)[0] for m in ms]
        tot = sum(pcs[1:], pcs[0])

        @pl.when(tot > 0)
        def _():
            gv = i * 128 + lane
            off = cnt
            for h in range(8):
                plsc.store_compressed(ckey.at[pl.ds(off, 16)], vs[h],
                                      mask=ms[h])
                plsc.store_compressed(cidx.at[pl.ds(off, 16)], gv + 16 * h,
                                      mask=ms[h])
                off = off + pcs[h]

        return cnt + tot

    ccnt = lax.fori_loop(0, _NPAD // 128, cscan, jnp.int32(0), unroll=2)

    def oscan(q, cnt):
        jv = q * 16 + lane
        ci = cidx[pl.ds(q * 16, 16)]
        own = (jv < ccnt) & (ci >= base) & (ci < base + _RPW)
        plsc.store_compressed(olist.at[pl.ds(cnt, 16)], jv, mask=own)
        return cnt + plsc.all_reduce_population_count(own)[0]

    ocnt = lax.fori_loop(0, 11, oscan, jnp.int32(0))

    cpx.wait()
    zrow = jnp.zeros((16,), jnp.float32)

    def zero_body(j, _):
        r = _sget(lidx, j)
        for cc in range(8):
            buf[r, pl.ds(cc * 16, 16)] = zrow
        return 0

    lax.fori_loop(0, zcnt, zero_body, 0)

    def repl_body(jj, _):
        j = _sget(olist, jj)
        kj = _sget(ckey, j)
        gi = _sget(cidx, j)
        kjv = jnp.full((16,), kj, jnp.int32)
        giv = jnp.full((16,), gi, jnp.int32)

        def rk(q, r):
            ck = ckey[pl.ds(q * 16, 16)]
            ci = cidx[pl.ds(q * 16, 16)]
            cmp = (ck > kjv) | ((ck == kjv) & (ci < giv))
            return r + plsc.all_reduce_population_count(cmp)[0]

        rank = lax.fori_loop(0, 11, rk, jnp.int32(0))
        rank = jnp.minimum(rank, _REPL_NUM - 1)
        pltpu.sync_copy(rand_hbm.at[pl.ds(rank * _D, _D)],
                        buf.at[gi - base])
        return 0

    lax.fori_loop(0, ocnt, repl_body, 0)
    pltpu.sync_copy(buf, out_hbm.at[pl.ds(base, _RPW)])


def _sc_apply(x, skey_pad, tvals, rand):
    mesh = plsc.VectorSubcoreMesh(core_axis_name="c", subcore_axis_name="s")
    return pl.kernel(
        _sc_apply_body,
        out_type=jax.ShapeDtypeStruct((_N, _D), jnp.float32),
        mesh=mesh,
        scratch_types=[
            pltpu.VMEM((_RPW, _D), jnp.float32),
            pltpu.VMEM((_NPAD,), jnp.int32),
            pltpu.VMEM((32,), jnp.int32),
            pltpu.VMEM((176,), jnp.int32),
            pltpu.VMEM((176,), jnp.int32),
            pltpu.VMEM((352,), jnp.int32),
            pltpu.VMEM((192,), jnp.int32),
            pltpu.SemaphoreType.DMA,
        ],
        compiler_params=pltpu.CompilerParams(needs_layout_passes=False),
    )(x, skey_pad, tvals, rand)


def kernel(x, edge_index, aug_type):
    del aug_type
    gum_pad, rand_flat = _op_constants()
    hist = _sc_bincount(edge_index)
    mask_i, skey, tvals = _tc_scores(hist, gum_pad)
    out = _sc_apply(x, skey, tvals, rand_flat)
    return out, mask_i
